# baseline (speedup 1.0000x reference)
import jax
import jax.numpy as jnp
from jax import lax
from jax.experimental import pallas as pl

_ROUTER_DIM = 80
_GATE_DIM = 16
_QUERY_DIM = _ROUTER_DIM + _GATE_DIM
_TOP_K = 4
_N_EXPERTS = 48
_TB = 1024

_DN = (((0,), (1,)), ((), ()))


def _router_body(q_ref, wr_ref, wg_ref, b_ref, rs_ref, gs_ref, idx_ref):
    q = q_ref[...]
    router = lax.dot_general(wr_ref[...], q[:, :_ROUTER_DIM], _DN,
                             preferred_element_type=jnp.float32)
    router = router + b_ref[:, 0:1]
    gb = lax.dot_general(wg_ref[...], q[:, _ROUTER_DIM:], _DN,
                         preferred_element_type=jnp.float32)
    gb = gb + b_ref[:, 1:2]

    tb = q.shape[0]
    iota_e = lax.broadcasted_iota(jnp.int32, (_N_EXPERTS, tb), 0)
    x = router
    scores, gsel, idxs = [], [], []
    for _ in range(_TOP_K):
        m = jnp.max(x, axis=0, keepdims=True)
        idx = jnp.min(jnp.where(x == m, iota_e, _N_EXPERTS),
                      axis=0, keepdims=True)
        sel = iota_e == idx
        scores.append(m)
        idxs.append(idx)
        gsel.append(jnp.sum(jnp.where(sel, gb, 0.0), axis=0, keepdims=True))
        x = jnp.where(sel, -jnp.inf, x)

    score = jnp.concatenate(scores, axis=0)
    gate_at = jnp.concatenate(gsel, axis=0)
    idx_ref[...] = jnp.concatenate(idxs, axis=0)
    rs_ref[...] = 1.0 / (1.0 + jnp.exp(-score))
    gs_ref[...] = 1.0 / (1.0 + jnp.exp(-(score + gate_at)))


@jax.jit
def kernel(query, key_pool):
    kp = key_pool[0]
    wr = kp[:_ROUTER_DIM, :]
    wg = kp[_ROUTER_DIM:_QUERY_DIM, :]
    biases = jnp.stack([kp[-4, :], kp[-3, :]], axis=1)
    n_tokens = query.shape[0]
    grid = (n_tokens // _TB,)
    out_shapes = (
        jax.ShapeDtypeStruct((_TOP_K, n_tokens), jnp.float32),
        jax.ShapeDtypeStruct((_TOP_K, n_tokens), jnp.float32),
        jax.ShapeDtypeStruct((_TOP_K, n_tokens), jnp.int32),
    )
    rs, gs, idx = pl.pallas_call(
        _router_body,
        grid=grid,
        in_specs=[
            pl.BlockSpec((_TB, _QUERY_DIM), lambda i: (i, 0)),
            pl.BlockSpec((_ROUTER_DIM, _N_EXPERTS), lambda i: (0, 0)),
            pl.BlockSpec((_GATE_DIM, _N_EXPERTS), lambda i: (0, 0)),
            pl.BlockSpec((_N_EXPERTS, 2), lambda i: (0, 0)),
        ],
        out_specs=(
            pl.BlockSpec((_TOP_K, _TB), lambda i: (0, i)),
            pl.BlockSpec((_TOP_K, _TB), lambda i: (0, i)),
            pl.BlockSpec((_TOP_K, _TB), lambda i: (0, i)),
        ),
        out_shape=out_shapes,
    )(query, wr, wg, biases)
    return rs, gs, idx


# speedup vs baseline: 1.1006x; 1.1006x over previous
import jax
import jax.numpy as jnp
from jax import lax
from jax.experimental import pallas as pl

_ROUTER_DIM = 80
_GATE_DIM = 16
_QUERY_DIM = _ROUTER_DIM + _GATE_DIM
_TOP_K = 4
_N_EXPERTS = 48
_TB = 2048

_DN = (((0,), (1,)), ((), ()))


def _router_body(q_ref, wr_ref, wg_ref, b_ref, rs_ref, gs_ref, idx_ref):
    q = q_ref[...]
    router = lax.dot_general(wr_ref[...], q[:, :_ROUTER_DIM], _DN,
                             preferred_element_type=jnp.float32)
    router = router + b_ref[:, 0:1]
    gb = lax.dot_general(wg_ref[...], q[:, _ROUTER_DIM:], _DN,
                         preferred_element_type=jnp.float32)
    gb = gb + b_ref[:, 1:2]

    tb = q.shape[0]
    iota_e = lax.broadcasted_iota(jnp.int32, (_N_EXPERTS, tb), 0)
    x = router
    scores, gsel, idxs = [], [], []
    for _ in range(_TOP_K):
        m = jnp.max(x, axis=0, keepdims=True)
        idx = jnp.min(jnp.where(x == m, iota_e, _N_EXPERTS),
                      axis=0, keepdims=True)
        sel = iota_e == idx
        scores.append(m)
        idxs.append(idx)
        gsel.append(jnp.sum(jnp.where(sel, gb, 0.0), axis=0, keepdims=True))
        x = jnp.where(sel, -jnp.inf, x)

    score = jnp.concatenate(scores, axis=0)
    gate_at = jnp.concatenate(gsel, axis=0)
    idx_ref[...] = jnp.concatenate(idxs, axis=0)
    rs_ref[...] = 1.0 / (1.0 + jnp.exp(-score))
    gs_ref[...] = 1.0 / (1.0 + jnp.exp(-(score + gate_at)))


@jax.jit
def kernel(query, key_pool):
    kp = key_pool[0]
    wr = kp[:_ROUTER_DIM, :]
    wg = kp[_ROUTER_DIM:_QUERY_DIM, :]
    biases = jnp.stack([kp[-4, :], kp[-3, :]], axis=1)
    n_tokens = query.shape[0]
    grid = (n_tokens // _TB,)
    out_shapes = (
        jax.ShapeDtypeStruct((_TOP_K, n_tokens), jnp.float32),
        jax.ShapeDtypeStruct((_TOP_K, n_tokens), jnp.float32),
        jax.ShapeDtypeStruct((_TOP_K, n_tokens), jnp.int32),
    )
    rs, gs, idx = pl.pallas_call(
        _router_body,
        grid=grid,
        in_specs=[
            pl.BlockSpec((_TB, _QUERY_DIM), lambda i: (i, 0)),
            pl.BlockSpec((_ROUTER_DIM, _N_EXPERTS), lambda i: (0, 0)),
            pl.BlockSpec((_GATE_DIM, _N_EXPERTS), lambda i: (0, 0)),
            pl.BlockSpec((_N_EXPERTS, 2), lambda i: (0, 0)),
        ],
        out_specs=(
            pl.BlockSpec((_TOP_K, _TB), lambda i: (0, i)),
            pl.BlockSpec((_TOP_K, _TB), lambda i: (0, i)),
            pl.BlockSpec((_TOP_K, _TB), lambda i: (0, i)),
        ),
        out_shape=out_shapes,
    )(query, wr, wg, biases)
    return rs.T, gs.T, idx.T


# speedup vs baseline: 1.1159x; 1.0139x over previous
import jax
import jax.numpy as jnp
from jax import lax
from jax.experimental import pallas as pl

_ROUTER_DIM = 80
_GATE_DIM = 16
_QUERY_DIM = _ROUTER_DIM + _GATE_DIM
_TOP_K = 4
_N_EXPERTS = 48
_TB = 4096

_DN = (((0,), (1,)), ((), ()))


def _router_body(q_ref, wr_ref, wg_ref, b_ref, rs_ref, gs_ref, idx_ref):
    q = q_ref[...]
    router = lax.dot_general(wr_ref[...], q[:, :_ROUTER_DIM], _DN,
                             preferred_element_type=jnp.float32)
    router = router + b_ref[:, 0:1]
    gb = lax.dot_general(wg_ref[...], q[:, _ROUTER_DIM:], _DN,
                         preferred_element_type=jnp.float32)
    gb = gb + b_ref[:, 1:2]

    tb = q.shape[0]
    iota_e = lax.broadcasted_iota(jnp.int32, (_N_EXPERTS, tb), 0)
    x = router
    scores, gsel, idxs = [], [], []
    for _ in range(_TOP_K):
        m = jnp.max(x, axis=0, keepdims=True)
        idx = jnp.min(jnp.where(x == m, iota_e, _N_EXPERTS),
                      axis=0, keepdims=True)
        sel = iota_e == idx
        scores.append(m)
        idxs.append(idx)
        gsel.append(jnp.sum(jnp.where(sel, gb, 0.0), axis=0, keepdims=True))
        x = jnp.where(sel, -jnp.inf, x)

    score = jnp.concatenate(scores, axis=0)
    gate_at = jnp.concatenate(gsel, axis=0)
    idx_ref[...] = jnp.concatenate(idxs, axis=0)
    rs_ref[...] = 1.0 / (1.0 + jnp.exp(-score))
    gs_ref[...] = 1.0 / (1.0 + jnp.exp(-(score + gate_at)))


@jax.jit
def kernel(query, key_pool):
    kp = key_pool[0]
    wr = kp[:_ROUTER_DIM, :]
    wg = kp[_ROUTER_DIM:_QUERY_DIM, :]
    biases = jnp.stack([kp[-4, :], kp[-3, :]], axis=1)
    n_tokens = query.shape[0]
    grid = (n_tokens // _TB,)
    out_shapes = (
        jax.ShapeDtypeStruct((_TOP_K, n_tokens), jnp.float32),
        jax.ShapeDtypeStruct((_TOP_K, n_tokens), jnp.float32),
        jax.ShapeDtypeStruct((_TOP_K, n_tokens), jnp.int32),
    )
    rs, gs, idx = pl.pallas_call(
        _router_body,
        grid=grid,
        in_specs=[
            pl.BlockSpec((_TB, _QUERY_DIM), lambda i: (i, 0)),
            pl.BlockSpec((_ROUTER_DIM, _N_EXPERTS), lambda i: (0, 0)),
            pl.BlockSpec((_GATE_DIM, _N_EXPERTS), lambda i: (0, 0)),
            pl.BlockSpec((_N_EXPERTS, 2), lambda i: (0, 0)),
        ],
        out_specs=(
            pl.BlockSpec((_TOP_K, _TB), lambda i: (0, i)),
            pl.BlockSpec((_TOP_K, _TB), lambda i: (0, i)),
            pl.BlockSpec((_TOP_K, _TB), lambda i: (0, i)),
        ),
        out_shape=out_shapes,
    )(query, wr, wg, biases)
    return rs.T, gs.T, idx.T
